# baseline (speedup 1.0000x reference)
import functools

import jax
import jax.numpy as jnp
from jax import lax
from jax.experimental import pallas as pl
from jax.experimental.pallas import tpu as pltpu
from jax.experimental.pallas import tpu_sc as plsc

_CHUNK = 128


def _sc_gather(idx3d, quad_table, lin2d):
    nw, c_per_w = idx3d.shape[0], idx3d.shape[1]
    n_chunks = nw * c_per_w
    k = quad_table.shape[1]
    info = plsc.get_sparse_core_info()
    nc = info.num_cores
    assert nc * info.num_subcores == nw
    n_grp = _CHUNK // 16

    mesh = plsc.VectorSubcoreMesh(core_axis_name="c", subcore_axis_name="s")

    @functools.partial(
        pl.kernel,
        mesh=mesh,
        compiler_params=pltpu.CompilerParams(use_tc_tiling_on_sc=False,
                                             needs_layout_passes=False),
        out_type=[
            jax.ShapeDtypeStruct((n_chunks, _CHUNK, k), jnp.float32),
            jax.ShapeDtypeStruct((nw, c_per_w, _CHUNK), jnp.float32),
        ],
        scratch_types=[
            pltpu.VMEM((c_per_w, _CHUNK), jnp.int32),
            pltpu.VMEM((c_per_w, _CHUNK), jnp.int32),
            pltpu.VMEM((c_per_w, _CHUNK, k), jnp.float32),
            pltpu.VMEM((c_per_w, _CHUNK, 16), jnp.float32),
            pltpu.VMEM((c_per_w, _CHUNK), jnp.float32),
            pltpu.SemaphoreType.DMA,
            pltpu.SemaphoreType.DMA,
        ],
    )
    def gather_kernel(idx_hbm, qt_hbm, lt_hbm, emb_out, lin_out,
                      idx_v, idxq_v, rows_v, lrows_v, lin_v, sem_q, sem_l):
        wid = lax.axis_index("s") * nc + lax.axis_index("c")
        base = wid * c_per_w
        pltpu.sync_copy(idx_hbm.at[wid], idx_v)

        def step(g, carry):
            for j in range(n_grp):
                v = idx_v[g, pl.ds(j * 16, 16)]
                idxq_v[g, pl.ds(j * 16, 16)] = lax.shift_right_logical(v, 4)
            cq = pltpu.async_copy(qt_hbm.at[idx_v.at[g]], rows_v.at[g], sem_q)
            cl = pltpu.async_copy(lt_hbm.at[idxq_v.at[g]], lrows_v.at[g], sem_l)
            cq.wait()
            cl.wait()
            for j in range(n_grp):
                v = idx_v[g, pl.ds(j * 16, 16)]
                lane = lax.bitwise_and(v, 15)
                row = lax.iota(jnp.int32, 16) + j * 16
                lin_v[g, pl.ds(j * 16, 16)] = plsc.load_gather(
                    lrows_v.at[g], [row, lane])
            return carry

        lax.fori_loop(0, c_per_w, step, 0)
        pltpu.sync_copy(rows_v, emb_out.at[pl.ds(base, c_per_w)])
        pltpu.sync_copy(lin_v, lin_out.at[wid])

    return gather_kernel(idx3d, quad_table, lin2d)


def _tc_head(emb, linv, s_mat, w1, b1, w2, b2, w3, cbias):
    b, d_in = emb.shape
    f = linv.shape[1]
    k = s_mat.shape[1]
    h1 = w1.shape[1]
    h2 = w2.shape[1]
    bb = 512
    grid = (b // bb,)

    def body(emb_ref, lin_ref, s_ref, w1_ref, b1_ref, w2_ref, b2_ref,
             w3_ref, cb_ref, out_ref):
        e = emb_ref[...]
        s = jnp.dot(e, s_ref[...], preferred_element_type=jnp.float32)
        quad = 0.5 * (jnp.sum(s * s, axis=1, keepdims=True)
                      - jnp.sum(e * e, axis=1, keepdims=True))
        lin = jnp.sum(lin_ref[...], axis=1, keepdims=True)
        h = jnp.dot(e, w1_ref[...], preferred_element_type=jnp.float32)
        h = jnp.maximum(h + b1_ref[...], 0.0)
        h = jnp.dot(h, w2_ref[...], preferred_element_type=jnp.float32)
        h = jnp.maximum(h + b2_ref[...], 0.0)
        ymlp = jnp.sum(h * w3_ref[...], axis=1, keepdims=True)
        z = cb_ref[...] + lin + quad + ymlp
        out_ref[...] = 1.0 / (1.0 + jnp.exp(-z))

    return pl.pallas_call(
        body,
        grid=grid,
        in_specs=[
            pl.BlockSpec((bb, d_in), lambda i: (i, 0)),
            pl.BlockSpec((bb, f), lambda i: (i, 0)),
            pl.BlockSpec((d_in, k), lambda i: (0, 0)),
            pl.BlockSpec((d_in, h1), lambda i: (0, 0)),
            pl.BlockSpec((1, h1), lambda i: (0, 0)),
            pl.BlockSpec((h1, h2), lambda i: (0, 0)),
            pl.BlockSpec((1, h2), lambda i: (0, 0)),
            pl.BlockSpec((1, h2), lambda i: (0, 0)),
            pl.BlockSpec((1, 1), lambda i: (0, 0)),
        ],
        out_specs=pl.BlockSpec((bb, 1), lambda i: (i, 0)),
        out_shape=jax.ShapeDtypeStruct((b, 1), jnp.float32),
    )(emb, linv, s_mat, w1, b1, w2, b2, w3, cbias)


def kernel(input, quad_table, lin_table, global_bias, W1, b1, W2, b2, W3, b3):
    b, f = input.shape
    r, k = quad_table.shape
    vocab = r // f
    offsets = jnp.arange(f, dtype=input.dtype) * vocab
    nw = 32
    idx3d = (input + offsets[None, :]).reshape(nw, -1, _CHUNK)
    lin2d = lin_table.reshape(-1, 16)
    emb3, lin3 = _sc_gather(idx3d, quad_table, lin2d)
    emb2d = emb3.reshape(b, f * k)
    linv = lin3.reshape(b, f)
    s_mat = jnp.tile(jnp.eye(k, dtype=jnp.float32), (f, 1))
    cbias = (global_bias[0] + b3[0]).reshape(1, 1)
    out = _tc_head(emb2d, linv, s_mat, W1, b1.reshape(1, -1), W2,
                   b2.reshape(1, -1), W3.reshape(1, -1), cbias)
    return out[:, 0]


# speedup vs baseline: 1.5360x; 1.5360x over previous
import functools
import math

import jax
import jax.numpy as jnp
from jax import lax
from jax.experimental import pallas as pl
from jax.experimental.pallas import tpu as pltpu
from jax.experimental.pallas import tpu_sc as plsc

_W = 8192


def _tc_pack(qt_t, lin_t):
    k, r = qt_t.shape
    g = math.ceil(r / _W)

    def body(q_ref, l_ref, q_out, l_out):
        y = jnp.transpose(q_ref[...])
        q_out[...] = jnp.concatenate(
            [y[s * 1024:(s + 1) * 1024, :] for s in range(8)], axis=1)
        z = l_ref[...]
        l_out[...] = jnp.concatenate(
            [z[:, c * 128:(c + 1) * 128] for c in range(64)], axis=0)

    return pl.pallas_call(
        body,
        grid=(g,),
        in_specs=[pl.BlockSpec((k, _W), lambda i: (0, i)),
                  pl.BlockSpec((1, _W), lambda i: (0, i))],
        out_specs=[pl.BlockSpec((1024, 128), lambda i: (i, 0)),
                   pl.BlockSpec((64, 128), lambda i: (i, 0))],
        out_shape=[jax.ShapeDtypeStruct((g * 1024, 128), jnp.float32),
                   jax.ShapeDtypeStruct((g * 64, 128), jnp.float32)],
    )(qt_t, lin_t)


def _sc_gather(idx3d, q128, l128, n_fields, n_bblk):
    nw, c_per_w = idx3d.shape[0], idx3d.shape[1]
    info = plsc.get_sparse_core_info()
    nc = info.num_cores
    assert nc * info.num_subcores == nw

    mesh = plsc.VectorSubcoreMesh(core_axis_name="c", subcore_axis_name="s")

    @functools.partial(
        pl.kernel,
        mesh=mesh,
        compiler_params=pltpu.CompilerParams(use_tc_tiling_on_sc=False,
                                             needs_layout_passes=False),
        out_type=[
            jax.ShapeDtypeStruct((n_fields, n_bblk, 16, 128), jnp.float32),
            jax.ShapeDtypeStruct((n_fields, n_bblk, 1, 128), jnp.float32),
        ],
        scratch_types=[
            pltpu.VMEM((c_per_w, 128), jnp.int32),
            pltpu.VMEM((c_per_w, 128), jnp.int32),
            pltpu.VMEM((c_per_w, 128), jnp.int32),
            pltpu.VMEM((2, 128, 128), jnp.float32),
            pltpu.VMEM((2, 128, 128), jnp.float32),
            pltpu.VMEM((16, 128), jnp.float32),
            pltpu.VMEM((1, 128), jnp.float32),
            pltpu.SemaphoreType.DMA((2,)),
            pltpu.SemaphoreType.DMA((2,)),
        ],
    )
    def gather_kernel(idx_hbm, q_hbm, l_hbm, emb_out, lin_out,
                      idx_v, qidx_v, lidx_v, qbuf, lbuf, ebuf, lvbuf,
                      sem_q, sem_l):
        wid = lax.axis_index("s") * nc + lax.axis_index("c")
        base = wid * c_per_w
        pltpu.sync_copy(idx_hbm.at[wid], idx_v)
        iota16 = lax.iota(jnp.int32, 16)

        def precomp(g, carry):
            for j in range(8):
                v = idx_v[g, pl.ds(j * 16, 16)]
                qidx_v[g, pl.ds(j * 16, 16)] = (
                    lax.shift_left(lax.shift_right_logical(v, 13), 10)
                    | lax.bitwise_and(v, 1023))
                lidx_v[g, pl.ds(j * 16, 16)] = lax.shift_right_logical(v, 7)
            return carry

        lax.fori_loop(0, c_per_w, precomp, 0)

        def start(g, slot):
            pltpu.async_copy(q_hbm.at[qidx_v.at[g]], qbuf.at[slot],
                             sem_q.at[slot])
            pltpu.async_copy(l_hbm.at[lidx_v.at[g]], lbuf.at[slot],
                             sem_l.at[slot])

        start(0, 0)

        def step(g, carry):
            slot = lax.bitwise_and(g, 1)

            @pl.when(g + 1 < c_per_w)
            def _():
                start(g + 1, 1 - slot)

            pltpu.make_async_copy(q_hbm.at[qidx_v.at[g]], qbuf.at[slot],
                                  sem_q.at[slot]).wait()
            pltpu.make_async_copy(l_hbm.at[lidx_v.at[g]], lbuf.at[slot],
                                  sem_l.at[slot]).wait()
            for j in range(8):
                v = idx_v[g, pl.ds(j * 16, 16)]
                s16 = lax.bitwise_and(lax.shift_right_logical(v, 10), 7) * 16
                rows = iota16 + j * 16
                for k in range(16):
                    ebuf[k, pl.ds(j * 16, 16)] = plsc.load_gather(
                        qbuf.at[slot], [rows, s16 + k])
                lvbuf[0, pl.ds(j * 16, 16)] = plsc.load_gather(
                    lbuf.at[slot], [rows, lax.bitwise_and(v, 127)])
            c = base + g
            f = lax.div(c, n_bblk)
            bb = lax.rem(c, n_bblk)
            pltpu.sync_copy(ebuf, emb_out.at[f, bb])
            pltpu.sync_copy(lvbuf, lin_out.at[f, bb])
            return carry

        lax.fori_loop(0, c_per_w, step, 0)

    return gather_kernel(idx3d, q128, l128)


def _tc_head(emb4, lin3, s_t, w1t, b1c, w2t, b2c, w3c, cbias):
    f, n_bblk = emb4.shape[0], emb4.shape[1]
    d_in = f * 16
    h1 = w1t.shape[0]
    h2 = w2t.shape[0]

    def body(emb_ref, lin_ref, s_ref, w1_ref, b1_ref, w2_ref, b2_ref,
             w3_ref, cb_ref, out_ref):
        x = emb_ref[...].reshape(d_in, 128)
        ksum = jnp.dot(s_ref[...], x, preferred_element_type=jnp.float32)
        sq_sum = jnp.sum(ksum * ksum, axis=0, keepdims=True)
        sum_sq = jnp.sum(x * x, axis=0, keepdims=True)
        quad = 0.5 * (sq_sum - sum_sq)
        lin = jnp.sum(lin_ref[...].reshape(f, 128), axis=0, keepdims=True)
        h = jnp.dot(w1_ref[...], x, preferred_element_type=jnp.float32)
        h = jnp.maximum(h + b1_ref[...], 0.0)
        h = jnp.dot(w2_ref[...], h, preferred_element_type=jnp.float32)
        h = jnp.maximum(h + b2_ref[...], 0.0)
        ymlp = jnp.sum(h * w3_ref[...], axis=0, keepdims=True)
        z = cb_ref[...] + lin + quad + ymlp
        out_ref[...] = (1.0 / (1.0 + jnp.exp(-z))).reshape(1, 1, 128)

    return pl.pallas_call(
        body,
        grid=(n_bblk,),
        in_specs=[
            pl.BlockSpec((f, 1, 16, 128), lambda i: (0, i, 0, 0)),
            pl.BlockSpec((f, 1, 1, 128), lambda i: (0, i, 0, 0)),
            pl.BlockSpec((16, d_in), lambda i: (0, 0)),
            pl.BlockSpec((h1, d_in), lambda i: (0, 0)),
            pl.BlockSpec((h1, 1), lambda i: (0, 0)),
            pl.BlockSpec((h2, h1), lambda i: (0, 0)),
            pl.BlockSpec((h2, 1), lambda i: (0, 0)),
            pl.BlockSpec((h2, 1), lambda i: (0, 0)),
            pl.BlockSpec((1, 1), lambda i: (0, 0)),
        ],
        out_specs=pl.BlockSpec((1, 1, 128), lambda i: (i, 0, 0)),
        out_shape=jax.ShapeDtypeStruct((n_bblk, 1, 128), jnp.float32),
    )(emb4, lin3, s_t, w1t, b1c, w2t, b2c, w3c, cbias)


def kernel(input, quad_table, lin_table, global_bias, W1, b1, W2, b2, W3, b3):
    b, f = input.shape
    r, k = quad_table.shape
    vocab = r // f
    nw = 32
    n_bblk = b // 128
    offsets = jnp.arange(f, dtype=input.dtype) * vocab
    idx_fm = input.T + offsets[:, None]
    idx3d = idx_fm.reshape(nw, -1, 128)
    q128, l128 = _tc_pack(quad_table.T, lin_table.T)
    emb4, lin3 = _sc_gather(idx3d, q128, l128, f, n_bblk)
    s_t = jnp.tile(jnp.eye(k, dtype=jnp.float32), (1, f))
    cbias = (global_bias[0] + b3[0]).reshape(1, 1)
    out = _tc_head(emb4, lin3, s_t, W1.T, b1.reshape(-1, 1), W2.T,
                   b2.reshape(-1, 1), W3, cbias)
    return out.reshape(b)


# speedup vs baseline: 3.4540x; 2.2487x over previous
import functools
import math

import jax
import jax.numpy as jnp
from jax import lax
from jax.experimental import pallas as pl
from jax.experimental.pallas import tpu as pltpu
from jax.experimental.pallas import tpu_sc as plsc

_W = 8192


def _tc_pack(qt_t, lin_t):
    k, r = qt_t.shape
    g = math.ceil(r / _W)

    def body(q_ref, l_ref, q_out, l_out):
        x = q_ref[...]
        xs = jnp.concatenate(
            [x[:, s * 1024:(s + 1) * 1024] for s in range(8)], axis=0)
        q_out[...] = jnp.transpose(xs)
        z = l_ref[...]
        l_out[...] = jnp.concatenate(
            [z[:, c * 128:(c + 1) * 128] for c in range(64)], axis=0)

    return pl.pallas_call(
        body,
        grid=(g,),
        in_specs=[pl.BlockSpec((k, _W), lambda i: (0, i)),
                  pl.BlockSpec((1, _W), lambda i: (0, i))],
        out_specs=[pl.BlockSpec((1024, 128), lambda i: (i, 0)),
                   pl.BlockSpec((64, 128), lambda i: (i, 0))],
        out_shape=[jax.ShapeDtypeStruct((g * 1024, 128), jnp.float32),
                   jax.ShapeDtypeStruct((g * 64, 128), jnp.float32)],
    )(qt_t, lin_t)


def _sc_gather(idx3d, q128, l128, n_fields, n_bblk):
    nw, c_per_w = idx3d.shape[0], idx3d.shape[1]
    info = plsc.get_sparse_core_info()
    nc = info.num_cores
    assert nc * info.num_subcores == nw

    mesh = plsc.VectorSubcoreMesh(core_axis_name="c", subcore_axis_name="s")

    @functools.partial(
        pl.kernel,
        mesh=mesh,
        compiler_params=pltpu.CompilerParams(use_tc_tiling_on_sc=False,
                                             needs_layout_passes=False),
        out_type=[
            jax.ShapeDtypeStruct((n_fields, n_bblk, 16, 128), jnp.float32),
            jax.ShapeDtypeStruct((n_fields, n_bblk, 1, 128), jnp.float32),
        ],
        scratch_types=[
            pltpu.VMEM((c_per_w, 128), jnp.int32),
            pltpu.VMEM((c_per_w, 128), jnp.int32),
            pltpu.VMEM((c_per_w, 128), jnp.int32),
            pltpu.VMEM((2, 128, 128), jnp.float32),
            pltpu.VMEM((2, 128, 128), jnp.float32),
            pltpu.VMEM((16, 128), jnp.float32),
            pltpu.VMEM((1, 128), jnp.float32),
            pltpu.SemaphoreType.DMA((2,)),
            pltpu.SemaphoreType.DMA((2,)),
        ],
    )
    def gather_kernel(idx_hbm, q_hbm, l_hbm, emb_out, lin_out,
                      idx_v, qidx_v, lidx_v, qbuf, lbuf, ebuf, lvbuf,
                      sem_q, sem_l):
        wid = lax.axis_index("s") * nc + lax.axis_index("c")
        base = wid * c_per_w
        pltpu.sync_copy(idx_hbm.at[wid], idx_v)
        iota16 = lax.iota(jnp.int32, 16)

        def precomp(g, carry):
            for j in range(8):
                v = idx_v[g, pl.ds(j * 16, 16)]
                qidx_v[g, pl.ds(j * 16, 16)] = (
                    lax.shift_left(lax.shift_right_logical(v, 13), 10)
                    | lax.bitwise_and(v, 1023))
                lidx_v[g, pl.ds(j * 16, 16)] = lax.shift_right_logical(v, 7)
            return carry

        lax.fori_loop(0, c_per_w, precomp, 0)

        def start(g, slot):
            pltpu.async_copy(q_hbm.at[qidx_v.at[g]], qbuf.at[slot],
                             sem_q.at[slot])
            pltpu.async_copy(l_hbm.at[lidx_v.at[g]], lbuf.at[slot],
                             sem_l.at[slot])

        start(0, 0)

        def step(g, carry):
            slot = lax.bitwise_and(g, 1)

            @pl.when(g + 1 < c_per_w)
            def _():
                start(g + 1, 1 - slot)

            pltpu.make_async_copy(q_hbm.at[qidx_v.at[g]], qbuf.at[slot],
                                  sem_q.at[slot]).wait()
            pltpu.make_async_copy(l_hbm.at[lidx_v.at[g]], lbuf.at[slot],
                                  sem_l.at[slot]).wait()
            for j in range(8):
                v = idx_v[g, pl.ds(j * 16, 16)]
                s16 = lax.bitwise_and(lax.shift_right_logical(v, 10), 7) * 16
                rows = iota16 + j * 16
                for k in range(16):
                    ebuf[k, pl.ds(j * 16, 16)] = plsc.load_gather(
                        qbuf.at[slot], [rows, s16 + k])
                lvbuf[0, pl.ds(j * 16, 16)] = plsc.load_gather(
                    lbuf.at[slot], [rows, lax.bitwise_and(v, 127)])
            c = base + g
            f = lax.div(c, n_bblk)
            bb = lax.rem(c, n_bblk)
            pltpu.sync_copy(ebuf, emb_out.at[f, bb])
            pltpu.sync_copy(lvbuf, lin_out.at[f, bb])
            return carry

        lax.fori_loop(0, c_per_w, step, 0)

    return gather_kernel(idx3d, q128, l128)


def _tc_head(emb4, lin3, s_t, w1t, b1c, w2t, b2c, w3c, cbias):
    f, n_bblk = emb4.shape[0], emb4.shape[1]
    d_in = f * 16
    h1 = w1t.shape[0]
    h2 = w2t.shape[0]

    def body(emb_ref, lin_ref, s_ref, w1_ref, b1_ref, w2_ref, b2_ref,
             w3_ref, cb_ref, out_ref):
        x = emb_ref[...].reshape(d_in, 128)
        ksum = jnp.dot(s_ref[...], x, preferred_element_type=jnp.float32)
        sq_sum = jnp.sum(ksum * ksum, axis=0, keepdims=True)
        sum_sq = jnp.sum(x * x, axis=0, keepdims=True)
        quad = 0.5 * (sq_sum - sum_sq)
        lin = jnp.sum(lin_ref[...].reshape(f, 128), axis=0, keepdims=True)
        h = jnp.dot(w1_ref[...], x, preferred_element_type=jnp.float32)
        h = jnp.maximum(h + b1_ref[...], 0.0)
        h = jnp.dot(w2_ref[...], h, preferred_element_type=jnp.float32)
        h = jnp.maximum(h + b2_ref[...], 0.0)
        ymlp = jnp.sum(h * w3_ref[...], axis=0, keepdims=True)
        z = cb_ref[...] + lin + quad + ymlp
        out_ref[...] = (1.0 / (1.0 + jnp.exp(-z))).reshape(1, 1, 128)

    return pl.pallas_call(
        body,
        grid=(n_bblk,),
        in_specs=[
            pl.BlockSpec((f, 1, 16, 128), lambda i: (0, i, 0, 0)),
            pl.BlockSpec((f, 1, 1, 128), lambda i: (0, i, 0, 0)),
            pl.BlockSpec((16, d_in), lambda i: (0, 0)),
            pl.BlockSpec((h1, d_in), lambda i: (0, 0)),
            pl.BlockSpec((h1, 1), lambda i: (0, 0)),
            pl.BlockSpec((h2, h1), lambda i: (0, 0)),
            pl.BlockSpec((h2, 1), lambda i: (0, 0)),
            pl.BlockSpec((h2, 1), lambda i: (0, 0)),
            pl.BlockSpec((1, 1), lambda i: (0, 0)),
        ],
        out_specs=pl.BlockSpec((1, 1, 128), lambda i: (i, 0, 0)),
        out_shape=jax.ShapeDtypeStruct((n_bblk, 1, 128), jnp.float32),
    )(emb4, lin3, s_t, w1t, b1c, w2t, b2c, w3c, cbias)


def kernel(input, quad_table, lin_table, global_bias, W1, b1, W2, b2, W3, b3):
    b, f = input.shape
    r, k = quad_table.shape
    vocab = r // f
    nw = 32
    n_bblk = b // 128
    offsets = jnp.arange(f, dtype=input.dtype) * vocab
    idx_fm = input.T + offsets[:, None]
    idx3d = idx_fm.reshape(nw, -1, 128)
    q128, l128 = _tc_pack(quad_table.T, lin_table.T)
    emb4, lin3 = _sc_gather(idx3d, q128, l128, f, n_bblk)
    s_t = jnp.tile(jnp.eye(k, dtype=jnp.float32), (1, f))
    cbias = (global_bias[0] + b3[0]).reshape(1, 1)
    out = _tc_head(emb4, lin3, s_t, W1.T, b1.reshape(-1, 1), W2.T,
                   b2.reshape(-1, 1), W3, cbias)
    return out.reshape(b)


# speedup vs baseline: 5.8166x; 1.6840x over previous
import functools
import math

import jax
import jax.numpy as jnp
from jax import lax
from jax.experimental import pallas as pl
from jax.experimental.pallas import tpu as pltpu
from jax.experimental.pallas import tpu_sc as plsc

_W = 32768


def _tc_pack(qt_t, lin_t):
    k, r = qt_t.shape
    g = math.ceil(r / _W)
    sw = _W // 8

    def body(q_ref, l_ref, q_out, l_out):
        x = q_ref[...]
        xs = jnp.concatenate(
            [x[:, s * sw:(s + 1) * sw] for s in range(8)], axis=0)
        q_out[...] = jnp.transpose(xs)
        z = l_ref[...]
        l_out[...] = jnp.concatenate(
            [z[:, c * 128:(c + 1) * 128] for c in range(_W // 128)], axis=0)

    return pl.pallas_call(
        body,
        grid=(g,),
        in_specs=[pl.BlockSpec((k, _W), lambda i: (0, i)),
                  pl.BlockSpec((1, _W), lambda i: (0, i))],
        out_specs=[pl.BlockSpec((_W // 8, 128), lambda i: (i, 0)),
                   pl.BlockSpec((_W // 128, 128), lambda i: (i, 0))],
        out_shape=[jax.ShapeDtypeStruct((g * _W // 8, 128), jnp.float32),
                   jax.ShapeDtypeStruct((g * _W // 128, 128), jnp.float32)],
        compiler_params=pltpu.CompilerParams(
            dimension_semantics=("arbitrary",)),
    )(qt_t, lin_t)


def _sc_gather(idx3d, q128, l128, n_fields, n_bblk):
    nw, c_per_w = idx3d.shape[0], idx3d.shape[1]
    info = plsc.get_sparse_core_info()
    nc = info.num_cores
    assert nc * info.num_subcores == nw

    mesh = plsc.VectorSubcoreMesh(core_axis_name="c", subcore_axis_name="s")

    @functools.partial(
        pl.kernel,
        mesh=mesh,
        compiler_params=pltpu.CompilerParams(use_tc_tiling_on_sc=False,
                                             needs_layout_passes=False),
        out_type=[
            jax.ShapeDtypeStruct((n_fields, n_bblk, 16, 128), jnp.float32),
            jax.ShapeDtypeStruct((n_fields, n_bblk, 1, 128), jnp.float32),
        ],
        scratch_types=[
            pltpu.VMEM((c_per_w, 128), jnp.int32),
            pltpu.VMEM((c_per_w, 128), jnp.int32),
            pltpu.VMEM((c_per_w, 128), jnp.int32),
            pltpu.VMEM((2, 128, 16), jnp.float32),
            pltpu.VMEM((2, 128, 16), jnp.float32),
            pltpu.VMEM((16, 128), jnp.float32),
            pltpu.VMEM((1, 128), jnp.float32),
            pltpu.SemaphoreType.DMA((2,)),
            pltpu.SemaphoreType.DMA((2,)),
        ],
    )
    def gather_kernel(idx_hbm, q_hbm, l_hbm, emb_out, lin_out,
                      idx_v, qidx_v, lidx_v, qbuf, lbuf, ebuf, lvbuf,
                      sem_q, sem_l):
        wid = lax.axis_index("s") * nc + lax.axis_index("c")
        base = wid * c_per_w
        pltpu.sync_copy(idx_hbm.at[wid], idx_v)
        iota16 = lax.iota(jnp.int32, 16)

        def precomp(g, carry):
            for j in range(8):
                v = idx_v[g, pl.ds(j * 16, 16)]
                qidx_v[g, pl.ds(j * 16, 16)] = (
                    lax.shift_left(lax.shift_right_logical(v, 15), 15)
                    | lax.shift_left(lax.bitwise_and(v, 4095), 3)
                    | lax.bitwise_and(lax.shift_right_logical(v, 12), 7))
                lidx_v[g, pl.ds(j * 16, 16)] = lax.shift_right_logical(v, 4)
            return carry

        lax.fori_loop(0, c_per_w, precomp, 0)

        def start(g, slot):
            pltpu.async_copy(q_hbm.at[qidx_v.at[g]], qbuf.at[slot],
                             sem_q.at[slot])
            pltpu.async_copy(l_hbm.at[lidx_v.at[g]], lbuf.at[slot],
                             sem_l.at[slot])

        start(0, 0)

        def step(g, carry):
            slot = lax.bitwise_and(g, 1)

            @pl.when(g + 1 < c_per_w)
            def _():
                start(g + 1, 1 - slot)

            pltpu.make_async_copy(q_hbm.at[qidx_v.at[g]], qbuf.at[slot],
                                  sem_q.at[slot]).wait()
            pltpu.make_async_copy(l_hbm.at[lidx_v.at[g]], lbuf.at[slot],
                                  sem_l.at[slot]).wait()
            for j in range(8):
                v = idx_v[g, pl.ds(j * 16, 16)]
                rows = iota16 + j * 16
                for k in range(16):
                    ebuf[k, pl.ds(j * 16, 16)] = plsc.load_gather(
                        qbuf.at[slot], [rows, iota16 * 0 + k])
                lvbuf[0, pl.ds(j * 16, 16)] = plsc.load_gather(
                    lbuf.at[slot], [rows, lax.bitwise_and(v, 15)])
            c = base + g
            f = lax.div(c, n_bblk)
            bb = lax.rem(c, n_bblk)
            pltpu.sync_copy(ebuf, emb_out.at[f, bb])
            pltpu.sync_copy(lvbuf, lin_out.at[f, bb])
            return carry

        lax.fori_loop(0, c_per_w, step, 0)

    return gather_kernel(idx3d, q128, l128)


def _tc_head(emb4, lin3, s_t, w1t, b1c, w2t, b2c, w3c, cbias):
    f, n_bblk = emb4.shape[0], emb4.shape[1]
    d_in = f * 16
    h1 = w1t.shape[0]
    h2 = w2t.shape[0]

    def body(emb_ref, lin_ref, s_ref, w1_ref, b1_ref, w2_ref, b2_ref,
             w3_ref, cb_ref, out_ref):
        x = emb_ref[...].reshape(d_in, 128)
        ksum = jnp.dot(s_ref[...], x, preferred_element_type=jnp.float32)
        sq_sum = jnp.sum(ksum * ksum, axis=0, keepdims=True)
        sum_sq = jnp.sum(x * x, axis=0, keepdims=True)
        quad = 0.5 * (sq_sum - sum_sq)
        lin = jnp.sum(lin_ref[...].reshape(f, 128), axis=0, keepdims=True)
        h = jnp.dot(w1_ref[...], x, preferred_element_type=jnp.float32)
        h = jnp.maximum(h + b1_ref[...], 0.0)
        h = jnp.dot(w2_ref[...], h, preferred_element_type=jnp.float32)
        h = jnp.maximum(h + b2_ref[...], 0.0)
        ymlp = jnp.sum(h * w3_ref[...], axis=0, keepdims=True)
        z = cb_ref[...] + lin + quad + ymlp
        out_ref[...] = (1.0 / (1.0 + jnp.exp(-z))).reshape(1, 1, 128)

    return pl.pallas_call(
        body,
        grid=(n_bblk,),
        in_specs=[
            pl.BlockSpec((f, 1, 16, 128), lambda i: (0, i, 0, 0)),
            pl.BlockSpec((f, 1, 1, 128), lambda i: (0, i, 0, 0)),
            pl.BlockSpec((16, d_in), lambda i: (0, 0)),
            pl.BlockSpec((h1, d_in), lambda i: (0, 0)),
            pl.BlockSpec((h1, 1), lambda i: (0, 0)),
            pl.BlockSpec((h2, h1), lambda i: (0, 0)),
            pl.BlockSpec((h2, 1), lambda i: (0, 0)),
            pl.BlockSpec((h2, 1), lambda i: (0, 0)),
            pl.BlockSpec((1, 1), lambda i: (0, 0)),
        ],
        out_specs=pl.BlockSpec((1, 1, 128), lambda i: (i, 0, 0)),
        out_shape=jax.ShapeDtypeStruct((n_bblk, 1, 128), jnp.float32),
    )(emb4, lin3, s_t, w1t, b1c, w2t, b2c, w3c, cbias)


def kernel(input, quad_table, lin_table, global_bias, W1, b1, W2, b2, W3, b3):
    b, f = input.shape
    r, k = quad_table.shape
    vocab = r // f
    nw = 32
    n_bblk = b // 128
    offsets = jnp.arange(f, dtype=input.dtype) * vocab
    idx_fm = input.T + offsets[:, None]
    idx3d = idx_fm.reshape(nw, -1, 128)
    q128, l128 = _tc_pack(quad_table.T, lin_table.T)
    emb4, lin3 = _sc_gather(idx3d, q128.reshape(-1, 16),
                            l128.reshape(-1, 16), f, n_bblk)
    s_t = jnp.tile(jnp.eye(k, dtype=jnp.float32), (1, f))
    cbias = (global_bias[0] + b3[0]).reshape(1, 1)
    out = _tc_head(emb4, lin3, s_t, W1.T, b1.reshape(-1, 1), W2.T,
                   b2.reshape(-1, 1), W3, cbias)
    return out.reshape(b)


# speedup vs baseline: 6.3151x; 1.0857x over previous
import functools
import math

import jax
import jax.numpy as jnp
from jax import lax
from jax.experimental import pallas as pl
from jax.experimental.pallas import tpu as pltpu
from jax.experimental.pallas import tpu_sc as plsc

_W = 65536


def _tc_pack(qt_t, lin_t):
    k, r = qt_t.shape
    g = math.ceil(r / _W)
    sw = _W // 8

    def body(q_ref, l_ref, q_out, l_out):
        x = q_ref[...]
        xs = jnp.concatenate(
            [x[:, s * sw:(s + 1) * sw] for s in range(8)], axis=0)
        q_out[...] = jnp.transpose(xs)
        z = l_ref[...]
        l_out[...] = jnp.concatenate(
            [z[:, c * 128:(c + 1) * 128] for c in range(_W // 128)], axis=0)

    return pl.pallas_call(
        body,
        grid=(g,),
        in_specs=[pl.BlockSpec((k, _W), lambda i: (0, i)),
                  pl.BlockSpec((1, _W), lambda i: (0, i))],
        out_specs=[pl.BlockSpec((_W // 8, 128), lambda i: (i, 0)),
                   pl.BlockSpec((_W // 128, 128), lambda i: (i, 0))],
        out_shape=[jax.ShapeDtypeStruct((g * _W // 8, 128), jnp.float32),
                   jax.ShapeDtypeStruct((g * _W // 128, 128), jnp.float32)],
        compiler_params=pltpu.CompilerParams(
            dimension_semantics=("arbitrary",)),
    )(qt_t, lin_t)


def _sc_gather(idx3d, q128, l128, n_fields, n_bblk):
    nw, c_per_w = idx3d.shape[0], idx3d.shape[1]
    info = plsc.get_sparse_core_info()
    nc = info.num_cores
    assert nc * info.num_subcores == nw

    mesh = plsc.VectorSubcoreMesh(core_axis_name="c", subcore_axis_name="s")

    @functools.partial(
        pl.kernel,
        mesh=mesh,
        compiler_params=pltpu.CompilerParams(use_tc_tiling_on_sc=False,
                                             needs_layout_passes=False),
        out_type=[
            jax.ShapeDtypeStruct((n_fields, n_bblk, 16, 128), jnp.float32),
            jax.ShapeDtypeStruct((n_fields, n_bblk, 1, 128), jnp.float32),
        ],
        scratch_types=[
            pltpu.VMEM((c_per_w, 128), jnp.int32),
            pltpu.VMEM((c_per_w, 128), jnp.int32),
            pltpu.VMEM((c_per_w, 128), jnp.int32),
            pltpu.VMEM((2, 128, 16), jnp.float32),
            pltpu.VMEM((2, 128, 16), jnp.float32),
            pltpu.VMEM((16, 128), jnp.float32),
            pltpu.VMEM((1, 128), jnp.float32),
            pltpu.SemaphoreType.DMA((2,)),
            pltpu.SemaphoreType.DMA((2,)),
        ],
    )
    def gather_kernel(idx_hbm, q_hbm, l_hbm, emb_out, lin_out,
                      idx_v, qidx_v, lidx_v, qbuf, lbuf, ebuf, lvbuf,
                      sem_q, sem_l):
        wid = lax.axis_index("s") * nc + lax.axis_index("c")
        base = wid * c_per_w
        pltpu.sync_copy(idx_hbm.at[wid], idx_v)
        iota16 = lax.iota(jnp.int32, 16)

        def precomp(g, carry):
            for j in range(8):
                v = idx_v[g, pl.ds(j * 16, 16)]
                qidx_v[g, pl.ds(j * 16, 16)] = (
                    lax.shift_left(lax.shift_right_logical(v, 16), 16)
                    | lax.shift_left(lax.bitwise_and(v, 8191), 3)
                    | lax.bitwise_and(lax.shift_right_logical(v, 13), 7))
                lidx_v[g, pl.ds(j * 16, 16)] = lax.shift_right_logical(v, 4)
            return carry

        lax.fori_loop(0, c_per_w, precomp, 0)

        def start(g, slot):
            pltpu.async_copy(q_hbm.at[qidx_v.at[g]], qbuf.at[slot],
                             sem_q.at[slot])
            pltpu.async_copy(l_hbm.at[lidx_v.at[g]], lbuf.at[slot],
                             sem_l.at[slot])

        start(0, 0)

        def step(g, carry):
            slot = lax.bitwise_and(g, 1)

            @pl.when(g + 1 < c_per_w)
            def _():
                start(g + 1, 1 - slot)

            pltpu.make_async_copy(q_hbm.at[qidx_v.at[g]], qbuf.at[slot],
                                  sem_q.at[slot]).wait()
            pltpu.make_async_copy(l_hbm.at[lidx_v.at[g]], lbuf.at[slot],
                                  sem_l.at[slot]).wait()
            for j in range(8):
                v = idx_v[g, pl.ds(j * 16, 16)]
                rows = iota16 + j * 16
                for k in range(16):
                    ebuf[k, pl.ds(j * 16, 16)] = plsc.load_gather(
                        qbuf.at[slot], [rows, iota16 * 0 + k])
                lvbuf[0, pl.ds(j * 16, 16)] = plsc.load_gather(
                    lbuf.at[slot], [rows, lax.bitwise_and(v, 15)])
            c = base + g
            f = lax.div(c, n_bblk)
            bb = lax.rem(c, n_bblk)
            pltpu.sync_copy(ebuf, emb_out.at[f, bb])
            pltpu.sync_copy(lvbuf, lin_out.at[f, bb])
            return carry

        lax.fori_loop(0, c_per_w, step, 0)

    return gather_kernel(idx3d, q128, l128)


def _tc_head(emb4, lin3, s_t, w1t, b1c, w2t, b2c, w3c, cbias):
    f, n_bblk = emb4.shape[0], emb4.shape[1]
    d_in = f * 16
    h1 = w1t.shape[0]
    h2 = w2t.shape[0]

    def body(emb_ref, lin_ref, s_ref, w1_ref, b1_ref, w2_ref, b2_ref,
             w3_ref, cb_ref, out_ref):
        x = emb_ref[...].reshape(d_in, 128)
        ksum = jnp.dot(s_ref[...], x, preferred_element_type=jnp.float32)
        sq_sum = jnp.sum(ksum * ksum, axis=0, keepdims=True)
        sum_sq = jnp.sum(x * x, axis=0, keepdims=True)
        quad = 0.5 * (sq_sum - sum_sq)
        lin = jnp.sum(lin_ref[...].reshape(f, 128), axis=0, keepdims=True)
        h = jnp.dot(w1_ref[...], x, preferred_element_type=jnp.float32)
        h = jnp.maximum(h + b1_ref[...], 0.0)
        h = jnp.dot(w2_ref[...], h, preferred_element_type=jnp.float32)
        h = jnp.maximum(h + b2_ref[...], 0.0)
        ymlp = jnp.sum(h * w3_ref[...], axis=0, keepdims=True)
        z = cb_ref[...] + lin + quad + ymlp
        out_ref[...] = (1.0 / (1.0 + jnp.exp(-z))).reshape(1, 1, 128)

    return pl.pallas_call(
        body,
        grid=(n_bblk,),
        in_specs=[
            pl.BlockSpec((f, 1, 16, 128), lambda i: (0, i, 0, 0)),
            pl.BlockSpec((f, 1, 1, 128), lambda i: (0, i, 0, 0)),
            pl.BlockSpec((16, d_in), lambda i: (0, 0)),
            pl.BlockSpec((h1, d_in), lambda i: (0, 0)),
            pl.BlockSpec((h1, 1), lambda i: (0, 0)),
            pl.BlockSpec((h2, h1), lambda i: (0, 0)),
            pl.BlockSpec((h2, 1), lambda i: (0, 0)),
            pl.BlockSpec((h2, 1), lambda i: (0, 0)),
            pl.BlockSpec((1, 1), lambda i: (0, 0)),
        ],
        out_specs=pl.BlockSpec((1, 1, 128), lambda i: (i, 0, 0)),
        out_shape=jax.ShapeDtypeStruct((n_bblk, 1, 128), jnp.float32),
    )(emb4, lin3, s_t, w1t, b1c, w2t, b2c, w3c, cbias)


def kernel(input, quad_table, lin_table, global_bias, W1, b1, W2, b2, W3, b3):
    b, f = input.shape
    r, k = quad_table.shape
    vocab = r // f
    nw = 32
    n_bblk = b // 128
    offsets = jnp.arange(f, dtype=input.dtype) * vocab
    idx_fm = input.T + offsets[:, None]
    idx3d = idx_fm.reshape(nw, -1, 128)
    q128, l128 = _tc_pack(quad_table.T, lin_table.T)
    emb4, lin3 = _sc_gather(idx3d, q128.reshape(-1, 16),
                            l128.reshape(-1, 16), f, n_bblk)
    s_t = jnp.tile(jnp.eye(k, dtype=jnp.float32), (1, f))
    cbias = (global_bias[0] + b3[0]).reshape(1, 1)
    out = _tc_head(emb4, lin3, s_t, W1.T, b1.reshape(-1, 1), W2.T,
                   b2.reshape(-1, 1), W3, cbias)
    return out.reshape(b)


# speedup vs baseline: 6.3872x; 1.0114x over previous
import functools
import math

import jax
import jax.numpy as jnp
from jax import lax
from jax.experimental import pallas as pl
from jax.experimental.pallas import tpu as pltpu
from jax.experimental.pallas import tpu_sc as plsc

_W = 131072


def _tc_pack(qt_t, lin_t):
    k, r = qt_t.shape
    g = math.ceil(r / _W)
    sw = _W // 8

    def body(q_ref, l_ref, q_out, l_out):
        x = q_ref[...]
        xs = jnp.concatenate(
            [x[:, s * sw:(s + 1) * sw] for s in range(8)], axis=0)
        q_out[...] = jnp.transpose(xs)
        z = l_ref[...]
        l_out[...] = jnp.concatenate(
            [z[:, c * 128:(c + 1) * 128] for c in range(_W // 128)], axis=0)

    return pl.pallas_call(
        body,
        grid=(g,),
        in_specs=[pl.BlockSpec((k, _W), lambda i: (0, i)),
                  pl.BlockSpec((1, _W), lambda i: (0, i))],
        out_specs=[pl.BlockSpec((_W // 8, 128), lambda i: (i, 0)),
                   pl.BlockSpec((_W // 128, 128), lambda i: (i, 0))],
        out_shape=[jax.ShapeDtypeStruct((g * _W // 8, 128), jnp.float32),
                   jax.ShapeDtypeStruct((g * _W // 128, 128), jnp.float32)],
        compiler_params=pltpu.CompilerParams(
            dimension_semantics=("arbitrary",)),
    )(qt_t, lin_t)


def _sc_gather(idx3d, q128, l128, n_fields, n_bblk):
    nw, c_per_w = idx3d.shape[0], idx3d.shape[1]
    info = plsc.get_sparse_core_info()
    nc = info.num_cores
    assert nc * info.num_subcores == nw

    mesh = plsc.VectorSubcoreMesh(core_axis_name="c", subcore_axis_name="s")

    @functools.partial(
        pl.kernel,
        mesh=mesh,
        compiler_params=pltpu.CompilerParams(use_tc_tiling_on_sc=False,
                                             needs_layout_passes=False),
        out_type=[
            jax.ShapeDtypeStruct((n_fields, n_bblk, 16, 128), jnp.float32),
            jax.ShapeDtypeStruct((n_fields, n_bblk, 1, 128), jnp.float32),
        ],
        scratch_types=[
            pltpu.VMEM((c_per_w, 128), jnp.int32),
            pltpu.VMEM((c_per_w, 128), jnp.int32),
            pltpu.VMEM((c_per_w, 128), jnp.int32),
            pltpu.VMEM((2, 128, 16), jnp.float32),
            pltpu.VMEM((2, 128, 16), jnp.float32),
            pltpu.VMEM((16, 128), jnp.float32),
            pltpu.VMEM((1, 128), jnp.float32),
            pltpu.SemaphoreType.DMA((2,)),
            pltpu.SemaphoreType.DMA((2,)),
        ],
    )
    def gather_kernel(idx_hbm, q_hbm, l_hbm, emb_out, lin_out,
                      idx_v, qidx_v, lidx_v, qbuf, lbuf, ebuf, lvbuf,
                      sem_q, sem_l):
        wid = lax.axis_index("s") * nc + lax.axis_index("c")
        base = wid * c_per_w
        pltpu.sync_copy(idx_hbm.at[wid], idx_v)
        iota16 = lax.iota(jnp.int32, 16)

        def precomp(g, carry):
            for j in range(8):
                v = idx_v[g, pl.ds(j * 16, 16)]
                qidx_v[g, pl.ds(j * 16, 16)] = (
                    lax.shift_left(lax.shift_right_logical(v, 17), 17)
                    | lax.shift_left(lax.bitwise_and(v, 16383), 3)
                    | lax.bitwise_and(lax.shift_right_logical(v, 14), 7))
                lidx_v[g, pl.ds(j * 16, 16)] = lax.shift_right_logical(v, 4)
            return carry

        lax.fori_loop(0, c_per_w, precomp, 0)

        def start(g, slot):
            pltpu.async_copy(q_hbm.at[qidx_v.at[g]], qbuf.at[slot],
                             sem_q.at[slot])
            pltpu.async_copy(l_hbm.at[lidx_v.at[g]], lbuf.at[slot],
                             sem_l.at[slot])

        start(0, 0)

        def step(g, carry):
            slot = lax.bitwise_and(g, 1)

            @pl.when(g + 1 < c_per_w)
            def _():
                start(g + 1, 1 - slot)

            pltpu.make_async_copy(q_hbm.at[qidx_v.at[g]], qbuf.at[slot],
                                  sem_q.at[slot]).wait()
            pltpu.make_async_copy(l_hbm.at[lidx_v.at[g]], lbuf.at[slot],
                                  sem_l.at[slot]).wait()
            for j in range(8):
                v = idx_v[g, pl.ds(j * 16, 16)]
                rows = iota16 + j * 16
                for k in range(16):
                    ebuf[k, pl.ds(j * 16, 16)] = plsc.load_gather(
                        qbuf.at[slot], [rows, iota16 * 0 + k])
                lvbuf[0, pl.ds(j * 16, 16)] = plsc.load_gather(
                    lbuf.at[slot], [rows, lax.bitwise_and(v, 15)])
            c = base + g
            f = lax.div(c, n_bblk)
            bb = lax.rem(c, n_bblk)
            pltpu.sync_copy(ebuf, emb_out.at[f, bb])
            pltpu.sync_copy(lvbuf, lin_out.at[f, bb])
            return carry

        lax.fori_loop(0, c_per_w, step, 0)

    return gather_kernel(idx3d, q128, l128)


def _tc_head(emb4, lin3, s_t, w1t, b1c, w2t, b2c, w3c, cbias):
    f, n_bblk = emb4.shape[0], emb4.shape[1]
    d_in = f * 16
    h1 = w1t.shape[0]
    h2 = w2t.shape[0]

    def body(emb_ref, lin_ref, s_ref, w1_ref, b1_ref, w2_ref, b2_ref,
             w3_ref, cb_ref, out_ref):
        x = emb_ref[...].reshape(d_in, 128)
        ksum = jnp.dot(s_ref[...], x, preferred_element_type=jnp.float32)
        sq_sum = jnp.sum(ksum * ksum, axis=0, keepdims=True)
        sum_sq = jnp.sum(x * x, axis=0, keepdims=True)
        quad = 0.5 * (sq_sum - sum_sq)
        lin = jnp.sum(lin_ref[...].reshape(f, 128), axis=0, keepdims=True)
        h = jnp.dot(w1_ref[...], x, preferred_element_type=jnp.float32)
        h = jnp.maximum(h + b1_ref[...], 0.0)
        h = jnp.dot(w2_ref[...], h, preferred_element_type=jnp.float32)
        h = jnp.maximum(h + b2_ref[...], 0.0)
        ymlp = jnp.sum(h * w3_ref[...], axis=0, keepdims=True)
        z = cb_ref[...] + lin + quad + ymlp
        out_ref[...] = (1.0 / (1.0 + jnp.exp(-z))).reshape(1, 1, 128)

    return pl.pallas_call(
        body,
        grid=(n_bblk,),
        in_specs=[
            pl.BlockSpec((f, 1, 16, 128), lambda i: (0, i, 0, 0)),
            pl.BlockSpec((f, 1, 1, 128), lambda i: (0, i, 0, 0)),
            pl.BlockSpec((16, d_in), lambda i: (0, 0)),
            pl.BlockSpec((h1, d_in), lambda i: (0, 0)),
            pl.BlockSpec((h1, 1), lambda i: (0, 0)),
            pl.BlockSpec((h2, h1), lambda i: (0, 0)),
            pl.BlockSpec((h2, 1), lambda i: (0, 0)),
            pl.BlockSpec((h2, 1), lambda i: (0, 0)),
            pl.BlockSpec((1, 1), lambda i: (0, 0)),
        ],
        out_specs=pl.BlockSpec((1, 1, 128), lambda i: (i, 0, 0)),
        out_shape=jax.ShapeDtypeStruct((n_bblk, 1, 128), jnp.float32),
    )(emb4, lin3, s_t, w1t, b1c, w2t, b2c, w3c, cbias)


def kernel(input, quad_table, lin_table, global_bias, W1, b1, W2, b2, W3, b3):
    b, f = input.shape
    r, k = quad_table.shape
    vocab = r // f
    nw = 32
    n_bblk = b // 128
    offsets = jnp.arange(f, dtype=input.dtype) * vocab
    idx_fm = input.T + offsets[:, None]
    idx3d = idx_fm.reshape(nw, -1, 128)
    q128, l128 = _tc_pack(quad_table.T, lin_table.T)
    emb4, lin3 = _sc_gather(idx3d, q128.reshape(-1, 16),
                            l128.reshape(-1, 16), f, n_bblk)
    s_t = jnp.tile(jnp.eye(k, dtype=jnp.float32), (1, f))
    cbias = (global_bias[0] + b3[0]).reshape(1, 1)
    out = _tc_head(emb4, lin3, s_t, W1.T, b1.reshape(-1, 1), W2.T,
                   b2.reshape(-1, 1), W3, cbias)
    return out.reshape(b)


# speedup vs baseline: 6.9639x; 1.0903x over previous
import functools
import math

import jax
import jax.numpy as jnp
from jax import lax
from jax.experimental import pallas as pl
from jax.experimental.pallas import tpu as pltpu
from jax.experimental.pallas import tpu_sc as plsc

_W = 131072


def _tc_pack(qt_t, lin_t):
    k, r = qt_t.shape
    g = math.ceil(r / _W)
    sw = _W // 8

    def body(q_ref, l_ref, q_out, l_out):
        x = q_ref[...]
        xs = jnp.concatenate(
            [x[:, s * sw:(s + 1) * sw] for s in range(8)], axis=0)
        q_out[...] = jnp.transpose(xs)
        z = l_ref[...]
        l_out[...] = jnp.concatenate(
            [z[:, c * 128:(c + 1) * 128] for c in range(_W // 128)], axis=0)

    return pl.pallas_call(
        body,
        grid=(g,),
        in_specs=[pl.BlockSpec((k, _W), lambda i: (0, i)),
                  pl.BlockSpec((1, _W), lambda i: (0, i))],
        out_specs=[pl.BlockSpec((_W // 8, 128), lambda i: (i, 0)),
                   pl.BlockSpec((_W // 128, 128), lambda i: (i, 0))],
        out_shape=[jax.ShapeDtypeStruct((g * _W // 8, 128), jnp.float32),
                   jax.ShapeDtypeStruct((g * _W // 128, 128), jnp.float32)],
        compiler_params=pltpu.CompilerParams(
            dimension_semantics=("arbitrary",)),
    )(qt_t, lin_t)


def _sc_gather(idx3d, q128, l128, n_fields, n_bblk):
    nw, c_per_w = idx3d.shape[0], idx3d.shape[1]
    info = plsc.get_sparse_core_info()
    nc = info.num_cores
    assert nc * info.num_subcores == nw

    mesh = plsc.VectorSubcoreMesh(core_axis_name="c", subcore_axis_name="s")

    @functools.partial(
        pl.kernel,
        mesh=mesh,
        compiler_params=pltpu.CompilerParams(use_tc_tiling_on_sc=False,
                                             needs_layout_passes=False),
        out_type=[
            jax.ShapeDtypeStruct((n_fields, n_bblk, 16, 128), jnp.float32),
            jax.ShapeDtypeStruct((n_fields, n_bblk, 1, 128), jnp.float32),
        ],
        scratch_types=[
            pltpu.VMEM((c_per_w, 128), jnp.int32),
            pltpu.VMEM((c_per_w, 128), jnp.int32),
            pltpu.VMEM((c_per_w, 128), jnp.int32),
            pltpu.VMEM((2, 128, 16), jnp.float32),
            pltpu.VMEM((2, 128, 16), jnp.float32),
            pltpu.VMEM((16, 128), jnp.float32),
            pltpu.VMEM((1, 128), jnp.float32),
            pltpu.SemaphoreType.DMA((2,)),
            pltpu.SemaphoreType.DMA((2,)),
        ],
    )
    def gather_kernel(idx_hbm, q_hbm, l_hbm, emb_out, lin_out,
                      idx_v, qidx_v, lidx_v, qbuf, lbuf, ebuf, lvbuf,
                      sem_q, sem_l):
        wid = lax.axis_index("s") * nc + lax.axis_index("c")
        base = wid * c_per_w
        pltpu.sync_copy(idx_hbm.at[wid], idx_v)
        iota16 = lax.iota(jnp.int32, 16)

        def precomp(g, carry):
            for j in range(8):
                v = idx_v[g, pl.ds(j * 16, 16)]
                qidx_v[g, pl.ds(j * 16, 16)] = (
                    lax.shift_left(lax.shift_right_logical(v, 17), 17)
                    | lax.shift_left(lax.bitwise_and(v, 16383), 3)
                    | lax.bitwise_and(lax.shift_right_logical(v, 14), 7))
                lidx_v[g, pl.ds(j * 16, 16)] = lax.shift_right_logical(v, 4)
            return carry

        lax.fori_loop(0, c_per_w, precomp, 0)

        def start(g, slot):
            pltpu.async_copy(q_hbm.at[qidx_v.at[g]], qbuf.at[slot],
                             sem_q.at[slot])
            pltpu.async_copy(l_hbm.at[lidx_v.at[g]], lbuf.at[slot],
                             sem_l.at[slot])

        start(0, 0)

        def step(g, carry):
            slot = lax.bitwise_and(g, 1)

            @pl.when(g + 1 < c_per_w)
            def _():
                start(g + 1, 1 - slot)

            pltpu.make_async_copy(q_hbm.at[qidx_v.at[g]], qbuf.at[slot],
                                  sem_q.at[slot]).wait()
            pltpu.make_async_copy(l_hbm.at[lidx_v.at[g]], lbuf.at[slot],
                                  sem_l.at[slot]).wait()
            for j in range(8):
                v = idx_v[g, pl.ds(j * 16, 16)]
                rows = iota16 + j * 16
                for k in range(16):
                    ebuf[k, pl.ds(j * 16, 16)] = plsc.load_gather(
                        qbuf.at[slot], [rows, iota16 * 0 + k])
                lvbuf[0, pl.ds(j * 16, 16)] = plsc.load_gather(
                    lbuf.at[slot], [rows, lax.bitwise_and(v, 15)])
            c = base + g
            f = lax.div(c, n_bblk)
            bb = lax.rem(c, n_bblk)
            pltpu.sync_copy(ebuf, emb_out.at[f, bb])
            pltpu.sync_copy(lvbuf, lin_out.at[f, bb])
            return carry

        lax.fori_loop(0, c_per_w, step, 0)

    return gather_kernel(idx3d, q128, l128)


def _tc_head(emb4, lin3, s_t, w1t, b1c, w2t, b2c, w3c, cbias):
    f, n_bblk = emb4.shape[0], emb4.shape[1]
    d_in = f * 16
    h1 = w1t.shape[0]
    h2 = w2t.shape[0]

    nb = 4

    def body(emb_ref, lin_ref, s_ref, w1_ref, b1_ref, w2_ref, b2_ref,
             w3_ref, cb_ref, out_ref):
        x = jnp.concatenate(
            [emb_ref[:, q].reshape(d_in, 128) for q in range(nb)],
            axis=1)
        ksum = jnp.dot(s_ref[...], x, preferred_element_type=jnp.float32)
        sq_sum = jnp.sum(ksum * ksum, axis=0, keepdims=True)
        sum_sq = jnp.sum(x * x, axis=0, keepdims=True)
        quad = 0.5 * (sq_sum - sum_sq)
        lin = jnp.concatenate(
            [jnp.sum(lin_ref[:, q, 0, :], axis=0, keepdims=True)
             for q in range(nb)], axis=1)
        h = jnp.dot(w1_ref[...], x, preferred_element_type=jnp.float32)
        h = jnp.maximum(h + b1_ref[...], 0.0)
        h = jnp.dot(w2_ref[...], h, preferred_element_type=jnp.float32)
        h = jnp.maximum(h + b2_ref[...], 0.0)
        ymlp = jnp.sum(h * w3_ref[...], axis=0, keepdims=True)
        z = cb_ref[...] + lin + quad + ymlp
        out_ref[...] = (1.0 / (1.0 + jnp.exp(-z))).reshape(1, 1, nb * 128)

    return pl.pallas_call(
        body,
        grid=(n_bblk // 4,),
        in_specs=[
            pl.BlockSpec((f, 4, 16, 128), lambda i: (0, i, 0, 0)),
            pl.BlockSpec((f, 4, 1, 128), lambda i: (0, i, 0, 0)),
            pl.BlockSpec((16, d_in), lambda i: (0, 0)),
            pl.BlockSpec((h1, d_in), lambda i: (0, 0)),
            pl.BlockSpec((h1, 1), lambda i: (0, 0)),
            pl.BlockSpec((h2, h1), lambda i: (0, 0)),
            pl.BlockSpec((h2, 1), lambda i: (0, 0)),
            pl.BlockSpec((h2, 1), lambda i: (0, 0)),
            pl.BlockSpec((1, 1), lambda i: (0, 0)),
        ],
        out_specs=pl.BlockSpec((1, 1, 512), lambda i: (i, 0, 0)),
        out_shape=jax.ShapeDtypeStruct((n_bblk // 4, 1, 512), jnp.float32),
    )(emb4, lin3, s_t, w1t, b1c, w2t, b2c, w3c, cbias)


def kernel(input, quad_table, lin_table, global_bias, W1, b1, W2, b2, W3, b3):
    b, f = input.shape
    r, k = quad_table.shape
    vocab = r // f
    nw = 32
    n_bblk = b // 128
    offsets = jnp.arange(f, dtype=input.dtype) * vocab
    idx_fm = input.T + offsets[:, None]
    idx3d = idx_fm.reshape(nw, -1, 128)
    q128, l128 = _tc_pack(quad_table.T, lin_table.T)
    emb4, lin3 = _sc_gather(idx3d, q128.reshape(-1, 16),
                            l128.reshape(-1, 16), f, n_bblk)
    s_t = jnp.tile(jnp.eye(k, dtype=jnp.float32), (1, f))
    cbias = (global_bias[0] + b3[0]).reshape(1, 1)
    out = _tc_head(emb4, lin3, s_t, W1.T, b1.reshape(-1, 1), W2.T,
                   b2.reshape(-1, 1), W3, cbias)
    return out.reshape(b)
